# baseline (speedup 1.0000x reference)
import jax
import jax.numpy as jnp
from jax.experimental import pallas as pl
from jax.experimental.pallas import tpu as pltpu

B, S, D, E = 4, 4096, 2048, 16
TOP_K = 2

CR = 512
NCH = (B * S) // CR
NSLOT = 8
G = 4
CPB = S // CR


def _router_kernel(x_hbm, w_ref, b_ref, tw_ref, ti_ref, aw_ref,
                   buf_ref, acc_ref, sems):
    def start(c, slot):
        pltpu.make_async_copy(
            x_hbm.at[pl.ds(c * CR, CR), :],
            buf_ref.at[pl.ds(slot * CR, CR), :],
            sems.at[slot]).start()

    def wait(c, slot):
        pltpu.make_async_copy(
            x_hbm.at[pl.ds(c * CR, CR), :],
            buf_ref.at[pl.ds(slot * CR, CR), :],
            sems.at[slot]).wait()

    acc_ref[...] = jnp.zeros((B, D), jnp.float32)

    for slot in range(NSLOT):
        start(slot, slot)

    def body(ii, carry):
        for half in range(2):
            base = ii * CPB + half * G
            for q in range(G):
                wait(base + q, half * G + q)
            rows = buf_ref[pl.ds(half * G * CR, G * CR), :]
            part = jnp.sum(rows, axis=0)
            acc_ref[pl.ds(ii, 1), :] = acc_ref[pl.ds(ii, 1), :] + part[None]
            for q in range(G):
                nxt = base + q + NSLOT

                @pl.when(nxt < NCH)
                def _prefetch():
                    start(nxt, half * G + q)
        return carry

    jax.lax.fori_loop(0, B, body, 0)

    q_pool = acc_ref[...] * (1.0 / S)
    logits = jnp.dot(q_pool, w_ref[...],
                     preferred_element_type=jnp.float32) + b_ref[0]
    m = jnp.max(logits, axis=-1, keepdims=True)
    ex = jnp.exp(logits - m)
    aw = ex / jnp.sum(ex, axis=-1, keepdims=True)
    aw_ref[...] = aw

    cols = jax.lax.broadcasted_iota(jnp.int32, (B, E), 1)
    i1 = jnp.argmax(aw, axis=-1).astype(jnp.int32)
    v1 = jnp.max(aw, axis=-1)
    masked = jnp.where(cols == i1[:, None], -jnp.inf, aw)
    i2 = jnp.argmax(masked, axis=-1).astype(jnp.int32)
    v2 = jnp.max(masked, axis=-1)
    norm = 1.0 / (v1 + v2 + 1e-10)
    tw_ref[...] = jnp.stack([v1 * norm, v2 * norm], axis=-1)
    ti_ref[...] = jnp.stack([i1, i2], axis=-1)


@jax.jit
def kernel(x_f, W, b):
    x2 = x_f.reshape(B * S, D)
    b2 = b.reshape(1, E)
    out = pl.pallas_call(
        _router_kernel,
        in_specs=[
            pl.BlockSpec(memory_space=pl.ANY),
            pl.BlockSpec(memory_space=pltpu.VMEM),
            pl.BlockSpec(memory_space=pltpu.VMEM),
        ],
        out_specs=[
            pl.BlockSpec(memory_space=pltpu.VMEM),
            pl.BlockSpec(memory_space=pltpu.VMEM),
            pl.BlockSpec(memory_space=pltpu.VMEM),
        ],
        out_shape=[
            jax.ShapeDtypeStruct((B, TOP_K), jnp.float32),
            jax.ShapeDtypeStruct((B, TOP_K), jnp.int32),
            jax.ShapeDtypeStruct((B, E), jnp.float32),
        ],
        scratch_shapes=[
            pltpu.VMEM((NSLOT * CR, D), jnp.float32),
            pltpu.VMEM((B, D), jnp.float32),
            pltpu.SemaphoreType.DMA((NSLOT,)),
        ],
    )(x2, W, b2)
    return tuple(out)


# speedup vs baseline: 1.0341x; 1.0341x over previous
import jax
import jax.numpy as jnp
from jax.experimental import pallas as pl
from jax.experimental.pallas import tpu as pltpu

B, S, D, E = 4, 4096, 2048, 16
TOP_K = 2

CR = 2048
NCH = (B * S) // CR
NSLOT = 2


def _router_kernel(x_hbm, w_ref, b_ref, tw_ref, ti_ref, aw_ref,
                   buf_ref, acc_ref, sems):
    def start(c, slot):
        pltpu.make_async_copy(
            x_hbm.at[pl.ds(c * CR, CR), :],
            buf_ref.at[slot],
            sems.at[slot]).start()

    def wait(c, slot):
        pltpu.make_async_copy(
            x_hbm.at[pl.ds(c * CR, CR), :],
            buf_ref.at[slot],
            sems.at[slot]).wait()

    start(0, 0)
    start(1, 1)

    def body(i, carry):
        for par in range(NSLOT):
            c = 2 * i + par
            wait(c, par)
            part = jnp.sum(buf_ref[par], axis=0)
            acc_ref[pl.ds(i, 1), :] = acc_ref[pl.ds(i, 1), :] + part[None]
            nxt = c + NSLOT

            @pl.when(nxt < NCH)
            def _prefetch():
                start(nxt, par)
        return carry

    acc_ref[...] = jnp.zeros((B, D), jnp.float32)
    jax.lax.fori_loop(0, B, body, 0)

    q_pool = acc_ref[...] * (1.0 / S)
    logits = jnp.dot(q_pool, w_ref[...],
                     preferred_element_type=jnp.float32) + b_ref[0]
    m = jnp.max(logits, axis=-1, keepdims=True)
    ex = jnp.exp(logits - m)
    aw = ex / jnp.sum(ex, axis=-1, keepdims=True)
    aw_ref[...] = aw

    cols = jax.lax.broadcasted_iota(jnp.int32, (B, E), 1)
    i1 = jnp.argmax(aw, axis=-1).astype(jnp.int32)
    v1 = jnp.max(aw, axis=-1)
    masked = jnp.where(cols == i1[:, None], -jnp.inf, aw)
    i2 = jnp.argmax(masked, axis=-1).astype(jnp.int32)
    v2 = jnp.max(masked, axis=-1)
    norm = 1.0 / (v1 + v2 + 1e-10)
    tw_ref[...] = jnp.stack([v1 * norm, v2 * norm], axis=-1)
    ti_ref[...] = jnp.stack([i1, i2], axis=-1)


@jax.jit
def kernel(x_f, W, b):
    x2 = x_f.reshape(B * S, D)
    b2 = b.reshape(1, E)
    out = pl.pallas_call(
        _router_kernel,
        in_specs=[
            pl.BlockSpec(memory_space=pl.ANY),
            pl.BlockSpec(memory_space=pltpu.VMEM),
            pl.BlockSpec(memory_space=pltpu.VMEM),
        ],
        out_specs=[
            pl.BlockSpec(memory_space=pltpu.VMEM),
            pl.BlockSpec(memory_space=pltpu.VMEM),
            pl.BlockSpec(memory_space=pltpu.VMEM),
        ],
        out_shape=[
            jax.ShapeDtypeStruct((B, TOP_K), jnp.float32),
            jax.ShapeDtypeStruct((B, TOP_K), jnp.int32),
            jax.ShapeDtypeStruct((B, E), jnp.float32),
        ],
        scratch_shapes=[
            pltpu.VMEM((NSLOT, CR, D), jnp.float32),
            pltpu.VMEM((B, D), jnp.float32),
            pltpu.SemaphoreType.DMA((NSLOT,)),
        ],
    )(x2, W, b2)
    return tuple(out)


# speedup vs baseline: 1.0729x; 1.0375x over previous
import jax
import jax.numpy as jnp
from jax.experimental import pallas as pl
from jax.experimental.pallas import tpu as pltpu

B, S, D, E = 4, 4096, 2048, 16
TOP_K = 2
CHUNK = 1024
NS = S // CHUNK


def _router_kernel(x_ref, w_ref, b_ref, tw_ref, ti_ref, aw_ref, acc_ref):
    bi = pl.program_id(0)
    si = pl.program_id(1)

    part = jnp.sum(x_ref[0], axis=0)

    @pl.when(si == 0)
    def _init():
        acc_ref[bi, :] = part

    @pl.when(si != 0)
    def _acc():
        acc_ref[bi, :] = acc_ref[bi, :] + part

    @pl.when((bi == B - 1) & (si == NS - 1))
    def _finalize():
        q_pool = acc_ref[...] * (1.0 / S)
        logits = jnp.dot(q_pool, w_ref[...],
                         preferred_element_type=jnp.float32) + b_ref[0]
        m = jnp.max(logits, axis=-1, keepdims=True)
        ex = jnp.exp(logits - m)
        aw = ex / jnp.sum(ex, axis=-1, keepdims=True)
        aw_ref[...] = aw

        cols = jax.lax.broadcasted_iota(jnp.int32, (B, E), 1)
        i1 = jnp.argmax(aw, axis=-1).astype(jnp.int32)
        v1 = jnp.max(aw, axis=-1)
        masked = jnp.where(cols == i1[:, None], -jnp.inf, aw)
        i2 = jnp.argmax(masked, axis=-1).astype(jnp.int32)
        v2 = jnp.max(masked, axis=-1)
        norm = 1.0 / (v1 + v2 + 1e-10)
        tw_ref[...] = jnp.stack([v1 * norm, v2 * norm], axis=-1)
        ti_ref[...] = jnp.stack([i1, i2], axis=-1)


@jax.jit
def kernel(x_f, W, b):
    b2 = b.reshape(1, E)
    out = pl.pallas_call(
        _router_kernel,
        grid=(B, NS),
        in_specs=[
            pl.BlockSpec((1, CHUNK, D), lambda bi, si: (bi, si, 0)),
            pl.BlockSpec((D, E), lambda bi, si: (0, 0)),
            pl.BlockSpec((1, E), lambda bi, si: (0, 0)),
        ],
        out_specs=[
            pl.BlockSpec((B, TOP_K), lambda bi, si: (0, 0)),
            pl.BlockSpec((B, TOP_K), lambda bi, si: (0, 0)),
            pl.BlockSpec((B, E), lambda bi, si: (0, 0)),
        ],
        out_shape=[
            jax.ShapeDtypeStruct((B, TOP_K), jnp.float32),
            jax.ShapeDtypeStruct((B, TOP_K), jnp.int32),
            jax.ShapeDtypeStruct((B, E), jnp.float32),
        ],
        scratch_shapes=[pltpu.VMEM((B, D), jnp.float32)],
    )(x_f, W, b2)
    return tuple(out)
